# baseline (speedup 1.0000x reference)
import jax
import jax.numpy as jnp
from jax.experimental import pallas as pl
from jax.experimental.pallas import tpu as pltpu

SEQ, D, HID, C = 512, 768, 128, 5
NENT, SPAN = 128, 16
H2 = 2 * HID
G8 = 8 * HID
NTOK = NENT * SPAN

FB = 32
KB = 8


def _pool_body(idx_ref, enc_ref, w_ref, b_ref, out_ref):
    proj = jnp.dot(enc_ref[...], w_ref[0])
    iota = jax.lax.broadcasted_iota(jnp.int32, (NTOK, SEQ), 1)
    onehot = (iota == idx_ref[0]).astype(jnp.float32)
    g = jnp.dot(onehot, proj) + b_ref[0]
    hs = []
    for d in range(2):
        o = d * 4 * HID
        gi = g[:, o + 0 * HID: o + 1 * HID]
        gg = g[:, o + 2 * HID: o + 3 * HID]
        go = g[:, o + 3 * HID: o + 4 * HID]
        c = jax.nn.sigmoid(gi) * jnp.tanh(gg)
        hs.append(jax.nn.sigmoid(go) * jnp.tanh(c))
    h = jnp.concatenate(hs, axis=1)
    out_ref[0] = jnp.max(h.reshape(NENT, SPAN, H2), axis=1)


def _pair_body(tt_ref, lhs_ref, ee_ref, out_ref):
    lhs = lhs_ref[0]
    ee = ee_ref[0]
    for f in range(FB):
        x = jnp.dot(lhs, tt_ref[0, f])
        out_ref[0, f] = jax.lax.dot_general(
            ee, x, (((1,), (1,)), ((), ())))


def _pred_body(pa_ref, t2_ref, out_ref):
    for kl in range(KB):
        hep = pa_ref[0, :, kl, :]
        tep = pa_ref[1, :, kl, :]
        for m in range(C):
            v = jax.lax.dot_general(
                t2_ref[m], hep, (((0,), (0,)), ((), ())))
            out_ref[kl, m] = jax.lax.dot_general(
                v, tep, (((0,), (0,)), ((), ())))


def kernel(encoder_output, holder_idxs, target_idxs, exp_idxs,
           Wh_f, bh_f, Wh_b, bh_b, Wt_f, bt_f, Wt_b, bt_b,
           We_f, be_f, We_b, be_b, T_he, T_te, T_cls):
    f32 = jnp.float32
    enc = encoder_output[0]
    idxs = jnp.stack([holder_idxs, target_idxs, exp_idxs])
    idxs = idxs.reshape(3, NTOK, 1)
    w_stack = jnp.stack([
        jnp.concatenate([Wh_f.T, Wh_b.T], axis=1),
        jnp.concatenate([Wt_f.T, Wt_b.T], axis=1),
        jnp.concatenate([We_f.T, We_b.T], axis=1)])
    b_stack = jnp.stack([
        jnp.concatenate([bh_f, bh_b]),
        jnp.concatenate([bt_f, bt_b]),
        jnp.concatenate([be_f, be_b])]).reshape(3, 1, G8)

    hte = pl.pallas_call(
        _pool_body,
        grid=(3,),
        in_specs=[
            pl.BlockSpec((1, NTOK, 1), lambda t: (t, 0, 0)),
            pl.BlockSpec((SEQ, D), lambda t: (0, 0)),
            pl.BlockSpec((1, D, G8), lambda t: (t, 0, 0)),
            pl.BlockSpec((1, 1, G8), lambda t: (t, 0, 0)),
        ],
        out_specs=pl.BlockSpec((1, NENT, H2), lambda t: (t, 0, 0)),
        out_shape=jax.ShapeDtypeStruct((3, NENT, H2), f32),
        compiler_params=pltpu.CompilerParams(
            dimension_semantics=("parallel",)),
    )(idxs, enc, w_stack, b_stack)

    tt = jnp.stack([T_he, T_te]).transpose(0, 2, 1, 3)

    pa = pl.pallas_call(
        _pair_body,
        grid=(2, H2 // FB),
        in_specs=[
            pl.BlockSpec((1, FB, H2, H2), lambda t, fb: (t, fb, 0, 0)),
            pl.BlockSpec((1, NENT, H2), lambda t, fb: (t, 0, 0)),
            pl.BlockSpec((1, NENT, H2), lambda t, fb: (2, 0, 0)),
        ],
        out_specs=pl.BlockSpec((1, FB, NENT, NENT), lambda t, fb: (t, fb, 0, 0)),
        out_shape=jax.ShapeDtypeStruct((2, H2, NENT, NENT), f32),
        compiler_params=pltpu.CompilerParams(
            dimension_semantics=("parallel", "parallel")),
    )(tt, hte, hte)

    t2 = T_cls.transpose(1, 2, 0)

    pred_t = pl.pallas_call(
        _pred_body,
        grid=(NENT // KB,),
        in_specs=[
            pl.BlockSpec((2, H2, KB, NENT), lambda kb: (0, 0, kb, 0)),
            pl.BlockSpec((C, H2, H2), lambda kb: (0, 0, 0)),
        ],
        out_specs=pl.BlockSpec((KB, C, NENT, NENT), lambda kb: (kb, 0, 0, 0)),
        out_shape=jax.ShapeDtypeStruct((NENT, C, NENT, NENT), f32),
        compiler_params=pltpu.CompilerParams(
            dimension_semantics=("parallel",)),
    )(pa, t2)

    return pred_t.transpose(2, 3, 0, 1)


# speedup vs baseline: 1.3880x; 1.3880x over previous
import jax
import jax.numpy as jnp
from jax.experimental import pallas as pl
from jax.experimental.pallas import tpu as pltpu

SEQ, D, HID, C = 512, 768, 128, 5
NENT, SPAN = 128, 16
H2 = 2 * HID
G8 = 8 * HID
NTOK = NENT * SPAN

FB = 8
KB = 8

_f32 = jnp.float32
_bf16 = jnp.bfloat16


def _sp(a):
    ah = a.astype(_bf16)
    al = (a - ah.astype(_f32)).astype(_bf16)
    return ah, al


def _d(a, b, dims):
    return jax.lax.dot_general(a, b, (dims, ((), ())),
                               preferred_element_type=_f32)


def _dot3(sa, sb, dims):
    ah, al = sa
    bh, bl = sb
    return _d(ah, bh, dims) + _d(ah, bl, dims) + _d(al, bh, dims)


def _pool_body(idx_ref, enc_ref, w_ref, b_ref, out_ref):
    proj = _dot3(_sp(enc_ref[...]), _sp(w_ref[0]), ((1,), (0,)))
    iota = jax.lax.broadcasted_iota(jnp.int32, (NTOK, SEQ), 1)
    onehot = (iota == idx_ref[0]).astype(_bf16)
    ph, pl_ = _sp(proj)
    g = (_d(onehot, ph, ((1,), (0,))) + _d(onehot, pl_, ((1,), (0,)))
         + b_ref[0])
    hs = []
    for d in range(2):
        o = d * 4 * HID
        gi = g[:, o + 0 * HID: o + 1 * HID]
        gg = g[:, o + 2 * HID: o + 3 * HID]
        go = g[:, o + 3 * HID: o + 4 * HID]
        c = jax.nn.sigmoid(gi) * jnp.tanh(gg)
        hs.append(jax.nn.sigmoid(go) * jnp.tanh(c))
    h = jnp.concatenate(hs, axis=1)
    out_ref[0] = jnp.max(h.reshape(NENT, SPAN, H2), axis=1)


def _pair_body(the_ref, tte_ref, hte_ref, out_ref):
    see = _sp(hte_ref[2])
    for t in range(2):
        tref = the_ref if t == 0 else tte_ref
        x = _dot3(_sp(hte_ref[t]), _sp(tref[...]), ((1,), (0,)))
        for f in range(FB):
            sxf = _sp(x[:, f * H2:(f + 1) * H2])
            out_ref[t, f] = _dot3(see, sxf, ((1,), (1,)))


def _pred_body(pa_ref, t2_ref, out_ref):
    shep = _sp(pa_ref[0])
    tep = pa_ref[1]
    steps = [_sp(tep[:, kl * NENT:(kl + 1) * NENT]) for kl in range(KB)]
    for m in range(C):
        v = _dot3(_sp(t2_ref[m]), shep, ((0,), (0,)))
        for kl in range(KB):
            svs = _sp(v[:, kl * NENT:(kl + 1) * NENT])
            out_ref[kl, m] = _dot3(svs, steps[kl], ((0,), (0,)))


def kernel(encoder_output, holder_idxs, target_idxs, exp_idxs,
           Wh_f, bh_f, Wh_b, bh_b, Wt_f, bt_f, Wt_b, bt_b,
           We_f, be_f, We_b, be_b, T_he, T_te, T_cls):
    enc = encoder_output[0]
    idxs = jnp.stack([holder_idxs, target_idxs, exp_idxs])
    idxs = idxs.reshape(3, NTOK, 1)
    w_stack = jnp.stack([
        jnp.concatenate([Wh_f.T, Wh_b.T], axis=1),
        jnp.concatenate([Wt_f.T, Wt_b.T], axis=1),
        jnp.concatenate([We_f.T, We_b.T], axis=1)])
    b_stack = jnp.stack([
        jnp.concatenate([bh_f, bh_b]),
        jnp.concatenate([bt_f, bt_b]),
        jnp.concatenate([be_f, be_b])]).reshape(3, 1, G8)

    hte = pl.pallas_call(
        _pool_body,
        grid=(3,),
        in_specs=[
            pl.BlockSpec((1, NTOK, 1), lambda t: (t, 0, 0)),
            pl.BlockSpec((SEQ, D), lambda t: (0, 0)),
            pl.BlockSpec((1, D, G8), lambda t: (t, 0, 0)),
            pl.BlockSpec((1, 1, G8), lambda t: (t, 0, 0)),
        ],
        out_specs=pl.BlockSpec((1, NENT, H2), lambda t: (t, 0, 0)),
        out_shape=jax.ShapeDtypeStruct((3, NENT, H2), _f32),
        compiler_params=pltpu.CompilerParams(
            dimension_semantics=("parallel",)),
    )(idxs, enc, w_stack, b_stack)

    the2 = T_he.reshape(H2, H2 * H2)
    tte2 = T_te.reshape(H2, H2 * H2)

    pa = pl.pallas_call(
        _pair_body,
        grid=(H2 // FB,),
        in_specs=[
            pl.BlockSpec((H2, FB * H2), lambda fb: (0, fb)),
            pl.BlockSpec((H2, FB * H2), lambda fb: (0, fb)),
            pl.BlockSpec((3, NENT, H2), lambda fb: (0, 0, 0)),
        ],
        out_specs=pl.BlockSpec((2, FB, NENT, NENT), lambda fb: (0, fb, 0, 0)),
        out_shape=jax.ShapeDtypeStruct((2, H2, NENT, NENT), _f32),
        compiler_params=pltpu.CompilerParams(
            dimension_semantics=("parallel",)),
    )(the2, tte2, hte)

    pa2 = pa.reshape(2, H2, NENT * NENT)
    t2 = T_cls.transpose(1, 2, 0)

    pred_t = pl.pallas_call(
        _pred_body,
        grid=(NENT // KB,),
        in_specs=[
            pl.BlockSpec((2, H2, KB * NENT), lambda kb: (0, 0, kb)),
            pl.BlockSpec((C, H2, H2), lambda kb: (0, 0, 0)),
        ],
        out_specs=pl.BlockSpec((KB, C, NENT, NENT), lambda kb: (kb, 0, 0, 0)),
        out_shape=jax.ShapeDtypeStruct((NENT, C, NENT, NENT), _f32),
        compiler_params=pltpu.CompilerParams(
            dimension_semantics=("parallel",)),
    )(pa2, t2)

    return pred_t.transpose(2, 3, 0, 1)


# speedup vs baseline: 1.4163x; 1.0204x over previous
import jax
import jax.numpy as jnp
from jax.experimental import pallas as pl
from jax.experimental.pallas import tpu as pltpu

SEQ, D, HID, C = 512, 768, 128, 5
NENT, SPAN = 128, 16
H2 = 2 * HID
H4 = 4 * HID
NTOK = NENT * SPAN

FB = 8
KB = 8

_f32 = jnp.float32
_bf16 = jnp.bfloat16


def _sp(a):
    ah = a.astype(_bf16)
    al = (a - ah.astype(_f32)).astype(_bf16)
    return ah, al


def _d(a, b, dims):
    return jax.lax.dot_general(a, b, (dims, ((), ())),
                               preferred_element_type=_f32)


def _dot3(sa, sb, dims):
    ah, al = sa
    bh, bl = sb
    return _d(ah, bh, dims) + _d(ah, bl, dims) + _d(al, bh, dims)


def _lstm_h(g):
    gi = g[:, 0 * HID:1 * HID]
    gg = g[:, 2 * HID:3 * HID]
    go = g[:, 3 * HID:4 * HID]
    return jax.nn.sigmoid(go) * jnp.tanh(jax.nn.sigmoid(gi) * jnp.tanh(gg))


def _pool_one(idx_ref, senc, wf_ref, bf_ref, wb_ref, bb_ref, out_ref):
    iota = jax.lax.broadcasted_iota(jnp.int32, (NTOK, SEQ), 1)
    onehot = (iota == idx_ref[...]).astype(_bf16)
    hs = []
    for w_ref, b_ref in ((wf_ref, bf_ref), (wb_ref, bb_ref)):
        proj = _dot3(senc, _sp(w_ref[...]), ((1,), (1,)))
        ph, pl_ = _sp(proj)
        g = (_d(onehot, ph, ((1,), (0,))) + _d(onehot, pl_, ((1,), (0,)))
             + b_ref[...])
        hs.append(_lstm_h(g))
    h = jnp.concatenate(hs, axis=1)
    out_ref[0] = jnp.max(h.reshape(NENT, SPAN, H2), axis=1)


def _pool_body(i0_ref, i1_ref, i2_ref, enc_ref,
               whf_ref, bhf_ref, whb_ref, bhb_ref,
               wtf_ref, btf_ref, wtb_ref, btb_ref,
               wef_ref, bef_ref, web_ref, beb_ref, out_ref):
    t = pl.program_id(0)
    senc = _sp(enc_ref[...])
    groups = (
        (i0_ref, whf_ref, bhf_ref, whb_ref, bhb_ref),
        (i1_ref, wtf_ref, btf_ref, wtb_ref, btb_ref),
        (i2_ref, wef_ref, bef_ref, web_ref, beb_ref),
    )
    for tt, (iref, wf, bf, wb, bb) in enumerate(groups):
        pl.when(t == tt)(
            lambda iref=iref, wf=wf, bf=bf, wb=wb, bb=bb:
                _pool_one(iref, senc, wf, bf, wb, bb, out_ref))


def _pair_body(the_ref, tte_ref, hte_ref, out_ref):
    see = _sp(hte_ref[2])
    for t in range(2):
        tref = the_ref if t == 0 else tte_ref
        x = _dot3(_sp(hte_ref[t]), _sp(tref[...]), ((1,), (0,)))
        for f in range(FB):
            sxf = _sp(x[:, f * H2:(f + 1) * H2])
            out_ref[t, f] = _dot3(see, sxf, ((1,), (1,)))


def _pred_body(pa_ref, t2_ref, out_ref):
    shep = _sp(pa_ref[0])
    tep = pa_ref[1]
    steps = [_sp(tep[:, kl * NENT:(kl + 1) * NENT]) for kl in range(KB)]
    for m in range(C):
        v = _dot3(_sp(t2_ref[m]), shep, ((0,), (0,)))
        for kl in range(KB):
            svs = _sp(v[:, kl * NENT:(kl + 1) * NENT])
            out_ref[kl, m] = _dot3(svs, steps[kl], ((0,), (0,)))


def kernel(encoder_output, holder_idxs, target_idxs, exp_idxs,
           Wh_f, bh_f, Wh_b, bh_b, Wt_f, bt_f, Wt_b, bt_b,
           We_f, be_f, We_b, be_b, T_he, T_te, T_cls):
    enc = encoder_output[0]
    i0 = holder_idxs.reshape(NTOK, 1)
    i1 = target_idxs.reshape(NTOK, 1)
    i2 = exp_idxs.reshape(NTOK, 1)
    bs = [b.reshape(1, H4) for b in
          (bh_f, bh_b, bt_f, bt_b, be_f, be_b)]

    full = lambda *shape: pl.BlockSpec(shape, lambda t, _s=len(shape): (0,) * _s)
    wspec = pl.BlockSpec((H4, D), lambda t: (0, 0))
    bspec = pl.BlockSpec((1, H4), lambda t: (0, 0))
    ispec = pl.BlockSpec((NTOK, 1), lambda t: (0, 0))

    hte = pl.pallas_call(
        _pool_body,
        grid=(3,),
        in_specs=[ispec, ispec, ispec,
                  pl.BlockSpec((SEQ, D), lambda t: (0, 0)),
                  wspec, bspec, wspec, bspec,
                  wspec, bspec, wspec, bspec,
                  wspec, bspec, wspec, bspec],
        out_specs=pl.BlockSpec((1, NENT, H2), lambda t: (t, 0, 0)),
        out_shape=jax.ShapeDtypeStruct((3, NENT, H2), _f32),
        compiler_params=pltpu.CompilerParams(
            dimension_semantics=("parallel",)),
    )(i0, i1, i2, enc,
      Wh_f, bs[0], Wh_b, bs[1],
      Wt_f, bs[2], Wt_b, bs[3],
      We_f, bs[4], We_b, bs[5])

    the2 = T_he.reshape(H2, H2 * H2)
    tte2 = T_te.reshape(H2, H2 * H2)

    pa = pl.pallas_call(
        _pair_body,
        grid=(H2 // FB,),
        in_specs=[
            pl.BlockSpec((H2, FB * H2), lambda fb: (0, fb)),
            pl.BlockSpec((H2, FB * H2), lambda fb: (0, fb)),
            pl.BlockSpec((3, NENT, H2), lambda fb: (0, 0, 0)),
        ],
        out_specs=pl.BlockSpec((2, FB, NENT, NENT), lambda fb: (0, fb, 0, 0)),
        out_shape=jax.ShapeDtypeStruct((2, H2, NENT, NENT), _f32),
        compiler_params=pltpu.CompilerParams(
            dimension_semantics=("parallel",)),
    )(the2, tte2, hte)

    pa2 = pa.reshape(2, H2, NENT * NENT)
    t2 = T_cls.transpose(1, 2, 0)

    pred_t = pl.pallas_call(
        _pred_body,
        grid=(NENT // KB,),
        in_specs=[
            pl.BlockSpec((2, H2, KB * NENT), lambda kb: (0, 0, kb)),
            pl.BlockSpec((C, H2, H2), lambda kb: (0, 0, 0)),
        ],
        out_specs=pl.BlockSpec((KB, C, NENT, NENT), lambda kb: (kb, 0, 0, 0)),
        out_shape=jax.ShapeDtypeStruct((NENT, C, NENT, NENT), _f32),
        compiler_params=pltpu.CompilerParams(
            dimension_semantics=("parallel",)),
    )(pa2, t2)

    return pred_t.transpose(2, 3, 0, 1)
